# Initial kernel scaffold; baseline (speedup 1.0000x reference)
import jax
import jax.numpy as jnp
from jax.experimental import pallas as pl


def kernel(x, edge_index, edge_type, W1, root1, b1, W2, root2, b2, W3, root3, b3):
    raise NotImplementedError("write your pallas kernel here")


# speedup vs baseline: 21.7341x; 21.7341x over previous
import functools

import jax
import jax.numpy as jnp
from jax import lax
from jax.experimental import pallas as pl
from jax.experimental.pallas import tpu as pltpu
from jax.experimental.pallas import tpu_sc as plsc

N = 10000
E = 320000
F = 128
H = 128
R = 8

NC = 2
NS = 16
NW = NC * NS
EPW = E // NW
EC = 80
NCHUNK = EPW // EC

CPAD = 81920
CPS = CPAD // NS

NPAD = 10240
NPS = NPAD // NS
DRAIN = 128
IC = 80

_MESH = dict(core_axis_name="c", subcore_axis_name="s")
_SC_PARAMS = pltpu.CompilerParams(needs_layout_passes=False)


def _wid():
    return lax.axis_index("c") * NS + lax.axis_index("s")


def _make_cnt_kernel():
    @functools.partial(
        pl.kernel,
        out_type=jax.ShapeDtypeStruct((CPAD,), jnp.float32),
        mesh=plsc.VectorSubcoreMesh(**_MESH),
        compiler_params=_SC_PARAMS,
        scratch_types=[
            pltpu.VMEM((EC,), jnp.int32),
            pltpu.VMEM((EC,), jnp.float32),
            pltpu.VMEM((CPS,), jnp.float32),
            pltpu.VMEM_SHARED((CPAD,), jnp.float32),
        ],
    )
    def cnt_kernel(cidx_hbm, zeros_hbm, inv_hbm, idx_v, ones_v, stg_v, cacc):
        c = lax.axis_index("c")
        s = lax.axis_index("s")
        pltpu.sync_copy(
            zeros_hbm.at[pl.ds(s * CPS, CPS)], cacc.at[pl.ds(s * CPS, CPS)]
        )
        for g in range(EC // 16):
            ones_v[pl.ds(g * 16, 16)] = jnp.ones((16,), jnp.float32)
        plsc.subcore_barrier()

        def chunk(k, carry):
            base = s * (E // NS) + k * EC
            pltpu.sync_copy(cidx_hbm.at[pl.ds(base, EC)], idx_v)
            pltpu.sync_copy(ones_v, cacc.at[idx_v], add=True)
            return carry

        lax.fori_loop(0, (E // NS) // EC, chunk, 0)
        plsc.subcore_barrier()

        @pl.when(c == 0)
        def _():
            pltpu.sync_copy(cacc.at[pl.ds(s * CPS, CPS)], stg_v)
            def inv(g, carry):
                sl = pl.ds(g * 16, 16)
                v = stg_v[sl]
                stg_v[sl] = 1.0 / jnp.maximum(v, 1.0)
                return carry
            lax.fori_loop(0, CPS // 16, inv, 0)
            pltpu.sync_copy(stg_v, inv_hbm.at[pl.ds(s * CPS, CPS)])

    return cnt_kernel


WC = 400


def _make_w_kernel():
    @functools.partial(
        pl.kernel,
        out_type=jax.ShapeDtypeStruct((E,), jnp.float32),
        mesh=plsc.VectorSubcoreMesh(**_MESH),
        compiler_params=_SC_PARAMS,
        scratch_types=[
            pltpu.VMEM((CPAD,), jnp.float32),
            pltpu.VMEM((WC,), jnp.int32),
            pltpu.VMEM((WC,), jnp.float32),
        ],
    )
    def w_kernel(cidx_hbm, inv_hbm, w_hbm, inv_v, ci_v, w_v):
        wid = _wid()
        pltpu.sync_copy(inv_hbm, inv_v)

        def chunk(k, carry):
            base = wid * EPW + k * WC
            pltpu.sync_copy(cidx_hbm.at[pl.ds(base, WC)], ci_v)
            def grp(g, carry2):
                sl = pl.ds(g * 16, 16)
                w_v[sl] = plsc.load_gather(inv_v, [ci_v[sl]])
                return carry2
            lax.fori_loop(0, WC // 16, grp, 0)
            pltpu.sync_copy(w_v, w_hbm.at[pl.ds(base, WC)])
            return carry

        lax.fori_loop(0, EPW // WC, chunk, 0)

    return w_kernel


def _make_edge_kernel():
    @functools.partial(
        pl.kernel,
        out_type=jax.ShapeDtypeStruct((NC, NPAD, H), jnp.float32),
        mesh=plsc.VectorSubcoreMesh(**_MESH),
        compiler_params=_SC_PARAMS,
        scratch_types=[
            pltpu.VMEM((EC,), jnp.int32),
            pltpu.VMEM((EC,), jnp.int32),
            pltpu.VMEM((EC,), jnp.float32),
            pltpu.VMEM((EC, H), jnp.float32),
            pltpu.VMEM((DRAIN, H), jnp.float32),
            pltpu.VMEM_SHARED((NPAD, H), jnp.float32),
            pltpu.SemaphoreType.DMA,
        ],
    )
    def edge_kernel(hcat_hbm, fidx_hbm, dst_hbm, w_hbm, zrows_hbm, out_hbm,
                    idx_v, dst_v, w_v, rows_v, stg_v, acc, sem):
        c = lax.axis_index("c")
        s = lax.axis_index("s")
        wid = c * NS + s

        @pl.when(c == 0)
        def _():
            for t in range(NPS // IC):
                @pl.when(s * NPS + t * IC + IC <= N)
                def _():
                    pltpu.sync_copy(
                        hcat_hbm.at[pl.ds(R * N + s * NPS + t * IC, IC)],
                        acc.at[pl.ds(s * NPS + t * IC, IC)],
                    )

        @pl.when(c != 0)
        def _():
            pltpu.sync_copy(zrows_hbm, acc.at[pl.ds(s * NPS, NPS)])

        plsc.subcore_barrier()

        def chunk(k, carry):
            base = wid * EPW + k * EC
            pltpu.sync_copy(fidx_hbm.at[pl.ds(base, EC)], idx_v)
            pltpu.sync_copy(dst_hbm.at[pl.ds(base, EC)], dst_v)
            pltpu.sync_copy(w_hbm.at[pl.ds(base, EC)], w_v)
            pltpu.async_copy(hcat_hbm.at[idx_v], rows_v, sem).wait()

            def row(i, carry2):
                wb = plsc.load_gather(w_v, [jnp.full((16,), i, jnp.int32)])
                for j in range(H // 16):
                    sl = pl.ds(j * 16, 16)
                    rows_v[i, sl] = rows_v[i, sl] * wb
                return carry2

            lax.fori_loop(0, EC, row, 0)
            pltpu.sync_copy(rows_v, acc.at[dst_v], add=True)
            return carry

        lax.fori_loop(0, NCHUNK, chunk, 0)
        plsc.subcore_barrier()

        for t in range(NPS // DRAIN):
            r0 = s * NPS + t * DRAIN
            pltpu.sync_copy(acc.at[pl.ds(r0, DRAIN)], stg_v)
            pltpu.sync_copy(stg_v, out_hbm.at[c, pl.ds(r0, DRAIN)])

    return edge_kernel


NB = 5
BN = N // NB


def _mm_body(x_ref, w_ref, b_ref, o_ref):
    r = pl.program_id(1)
    y = jnp.dot(x_ref[...], w_ref[0], preferred_element_type=jnp.float32)
    o_ref[...] = y + jnp.where(r == R, 1.0, 0.0) * b_ref[...]


def _matmul(x, wcat, b):
    return pl.pallas_call(
        _mm_body,
        grid=(NB, R + 1),
        in_specs=[
            pl.BlockSpec((BN, F), lambda n, r: (n, 0)),
            pl.BlockSpec((1, F, H), lambda n, r: (r, 0, 0)),
            pl.BlockSpec((1, H), lambda n, r: (0, 0)),
        ],
        out_specs=pl.BlockSpec((BN, H), lambda n, r: (r * NB + n, 0)),
        out_shape=jax.ShapeDtypeStruct(((R + 1) * N, H), jnp.float32),
    )(x, wcat, b)


def _comb_body(p_ref, o_ref):
    o_ref[...] = jnp.maximum(p_ref[0] + p_ref[1], 0.0)


def _combine(parts):
    return pl.pallas_call(
        _comb_body,
        grid=(NB,),
        in_specs=[pl.BlockSpec((NC, BN, H), lambda n: (0, n, 0))],
        out_specs=pl.BlockSpec((BN, H), lambda n: (n, 0)),
        out_shape=jax.ShapeDtypeStruct((N, H), jnp.float32),
    )(parts)


def kernel(x, edge_index, edge_type, W1, root1, b1, W2, root2, b2, W3, root3, b3):
    src = edge_index[0]
    dst = edge_index[1]
    fidx = edge_type * N + src
    cidx = dst * R + edge_type
    zeros_c = jnp.zeros((CPAD,), jnp.float32)
    zrows = jnp.zeros((NPS, H), jnp.float32)

    inv = _make_cnt_kernel()(cidx, zeros_c)
    w = _make_w_kernel()(cidx, inv)

    edge_k = _make_edge_kernel()
    h = x
    for (Wr, rootr, br) in ((W1, root1, b1), (W2, root2, b2), (W3, root3, b3)):
        wcat = jnp.concatenate([Wr, rootr[None]], axis=0)
        hcat = _matmul(h, wcat, br.reshape(1, H))
        parts = edge_k(hcat, fidx, dst, w, zrows)
        h = _combine(parts)
    return h


# speedup vs baseline: 39.2619x; 1.8065x over previous
import functools

import jax
import jax.numpy as jnp
from jax import lax
from jax.experimental import pallas as pl
from jax.experimental.pallas import tpu as pltpu
from jax.experimental.pallas import tpu_sc as plsc

N = 10000
E = 320000
F = 128
H = 128
R = 8

NC = 2
NS = 16
NW = NC * NS
EPW = E // NW
EC = 80
NCHUNK = EPW // EC

CPAD = 81920
CPS = CPAD // NS

NPAD = 10240
NPS = NPAD // NS
DRAIN = 128
IC = 80

_MESH = dict(core_axis_name="c", subcore_axis_name="s")
_SC_PARAMS = pltpu.CompilerParams(needs_layout_passes=False)


def _wid():
    return lax.axis_index("c") * NS + lax.axis_index("s")


CCH = (E // NS) // EC


def _make_cnt_kernel():
    @functools.partial(
        pl.kernel,
        out_type=jax.ShapeDtypeStruct((CPAD,), jnp.float32),
        mesh=plsc.VectorSubcoreMesh(**_MESH),
        compiler_params=_SC_PARAMS,
        scratch_types=[
            pltpu.VMEM((CCH, EC), jnp.int32),
            pltpu.VMEM((EC,), jnp.float32),
            pltpu.VMEM((CPS,), jnp.float32),
            pltpu.VMEM_SHARED((CPAD,), jnp.float32),
        ],
    )
    def cnt_kernel(cidx_hbm, zeros_hbm, inv_hbm, idx_v, ones_v, stg_v, cacc):
        c = lax.axis_index("c")
        s = lax.axis_index("s")
        pltpu.sync_copy(
            zeros_hbm.at[pl.ds(s * CPS, CPS)], cacc.at[pl.ds(s * CPS, CPS)]
        )
        pltpu.sync_copy(cidx_hbm.at[s], idx_v)
        for g in range(EC // 16):
            ones_v[pl.ds(g * 16, 16)] = jnp.ones((16,), jnp.float32)
        plsc.subcore_barrier()

        def chunk(k, carry):
            pltpu.sync_copy(ones_v, cacc.at[idx_v.at[k]], add=True)
            return carry

        lax.fori_loop(0, CCH, chunk, 0)
        plsc.subcore_barrier()

        @pl.when(c == 0)
        def _():
            pltpu.sync_copy(cacc.at[pl.ds(s * CPS, CPS)], stg_v)
            def inv(g, carry):
                sl = pl.ds(g * 16, 16)
                v = stg_v[sl]
                stg_v[sl] = 1.0 / jnp.maximum(v, 1.0)
                return carry
            lax.fori_loop(0, CPS // 16, inv, 0)
            pltpu.sync_copy(stg_v, inv_hbm.at[pl.ds(s * CPS, CPS)])

    return cnt_kernel


WC = 400


def _make_w_kernel():
    @functools.partial(
        pl.kernel,
        out_type=jax.ShapeDtypeStruct((E,), jnp.float32),
        mesh=plsc.VectorSubcoreMesh(**_MESH),
        compiler_params=_SC_PARAMS,
        scratch_types=[
            pltpu.VMEM((CPAD,), jnp.float32),
            pltpu.VMEM((WC,), jnp.int32),
            pltpu.VMEM((WC,), jnp.float32),
        ],
    )
    def w_kernel(cidx_hbm, inv_hbm, w_hbm, inv_v, ci_v, w_v):
        wid = _wid()
        pltpu.sync_copy(inv_hbm, inv_v)

        def chunk(k, carry):
            base = wid * EPW + k * WC
            pltpu.sync_copy(cidx_hbm.at[pl.ds(base, WC)], ci_v)
            def grp(g, carry2):
                sl = pl.ds(g * 16, 16)
                w_v[sl] = plsc.load_gather(inv_v, [ci_v[sl]])
                return carry2
            lax.fori_loop(0, WC // 16, grp, 0)
            pltpu.sync_copy(w_v, w_hbm.at[pl.ds(base, WC)])
            return carry

        lax.fori_loop(0, EPW // WC, chunk, 0)

    return w_kernel


def _make_edge_kernel():
    @functools.partial(
        pl.kernel,
        out_type=jax.ShapeDtypeStruct((NC, NPAD, H), jnp.float32),
        mesh=plsc.VectorSubcoreMesh(**_MESH),
        compiler_params=_SC_PARAMS,
        scratch_types=[
            pltpu.VMEM((2, EC), jnp.int32),
            pltpu.VMEM((2, EC), jnp.int32),
            pltpu.VMEM((EC,), jnp.float32),
            pltpu.VMEM((EC,), jnp.float32),
            pltpu.VMEM((EC, H), jnp.float32),
            pltpu.VMEM((EC, H), jnp.float32),
            pltpu.VMEM_SHARED((NPAD, H), jnp.float32),
            pltpu.SemaphoreType.DMA,
            pltpu.SemaphoreType.DMA,
            pltpu.SemaphoreType.DMA,
            pltpu.SemaphoreType.DMA,
            pltpu.SemaphoreType.DMA,
            pltpu.SemaphoreType.DMA,
        ],
    )
    def edge_kernel(hcat_hbm, meta_hbm, w_hbm, zrows_hbm, out_hbm,
                    m0_v, m1_v, w0_v, w1_v, rows0_v, rows1_v, acc,
                    gsem0, gsem1, wsem0, wsem1, msem0, msem1):
        c = lax.axis_index("c")
        s = lax.axis_index("s")
        wid = c * NS + s

        @pl.when(c == 0)
        def _():
            for t in range(NPS // IC):
                @pl.when(s * NPS + t * IC + IC <= N)
                def _():
                    pltpu.sync_copy(
                        hcat_hbm.at[pl.ds(R * N + s * NPS + t * IC, IC)],
                        acc.at[pl.ds(s * NPS + t * IC, IC)],
                    )

        @pl.when(c != 0)
        def _():
            pltpu.sync_copy(zrows_hbm, acc.at[pl.ds(s * NPS, NPS)])

        plsc.subcore_barrier()

        rbufs = (rows0_v, rows1_v)
        gsems = (gsem0, gsem1)
        wbufs = (w0_v, w1_v)
        wsems = (wsem0, wsem1)
        mbufs = (m0_v, m1_v)
        msems = (msem0, msem1)

        def fire_meta(k, b):
            pltpu.async_copy(meta_hbm.at[wid, k], mbufs[b], msems[b])

        def wait_meta(k, b):
            pltpu.make_async_copy(meta_hbm.at[wid, k], mbufs[b], msems[b]).wait()

        def fire_rows(k, b):
            pltpu.async_copy(hcat_hbm.at[mbufs[b].at[0]], rbufs[b], gsems[b])
            pltpu.async_copy(w_hbm.at[wid, k], wbufs[b], wsems[b])

        def process(k, b):
            pltpu.make_async_copy(
                hcat_hbm.at[mbufs[b].at[0]], rbufs[b], gsems[b]
            ).wait()
            pltpu.make_async_copy(w_hbm.at[wid, k], wbufs[b], wsems[b]).wait()
            buf = rbufs[b]
            wb_ref = wbufs[b]

            def row(i, carry2):
                wb = plsc.load_gather(wb_ref, [jnp.full((16,), i, jnp.int32)])
                for j in range(H // 16):
                    sl = pl.ds(j * 16, 16)
                    buf[i, sl] = buf[i, sl] * wb
                return carry2

            lax.fori_loop(0, EC, row, 0)
            pltpu.sync_copy(buf, acc.at[mbufs[b].at[1]], add=True)

        fire_meta(0, 0)
        fire_meta(1, 1)
        wait_meta(0, 0)
        fire_rows(0, 0)

        def pair(t, carry):
            for b in range(2):
                kk = t * 2 + b
                b2 = (b + 1) % 2

                @pl.when(kk < NCHUNK)
                def _():
                    @pl.when(kk + 1 < NCHUNK)
                    def _():
                        wait_meta(kk + 1, b2)
                        fire_rows(kk + 1, b2)
                    process(kk, b)

                    @pl.when(kk + 2 < NCHUNK)
                    def _():
                        fire_meta(kk + 2, b)
            return carry

        lax.fori_loop(0, (NCHUNK + 1) // 2, pair, 0)
        plsc.subcore_barrier()

        for t in range(NPS // DRAIN):
            r0 = s * NPS + t * DRAIN
            pltpu.sync_copy(acc.at[pl.ds(r0, DRAIN)], out_hbm.at[c, pl.ds(r0, DRAIN)])

    return edge_kernel


NB = 5
BN = N // NB


def _mm_body(x_ref, w_ref, b_ref, o_ref):
    r = pl.program_id(1)
    y = jnp.dot(x_ref[...], w_ref[0], preferred_element_type=jnp.float32)
    o_ref[...] = y + jnp.where(r == R, 1.0, 0.0) * b_ref[...]


def _matmul(x, wcat, b):
    return pl.pallas_call(
        _mm_body,
        grid=(NB, R + 1),
        in_specs=[
            pl.BlockSpec((BN, F), lambda n, r: (n, 0)),
            pl.BlockSpec((1, F, H), lambda n, r: (r, 0, 0)),
            pl.BlockSpec((1, H), lambda n, r: (0, 0)),
        ],
        out_specs=pl.BlockSpec((BN, H), lambda n, r: (r * NB + n, 0)),
        out_shape=jax.ShapeDtypeStruct(((R + 1) * N, H), jnp.float32),
    )(x, wcat, b)


def _comb_body(p_ref, o_ref):
    o_ref[...] = jnp.maximum(p_ref[0] + p_ref[1], 0.0)


def _combine(parts):
    return pl.pallas_call(
        _comb_body,
        grid=(NB,),
        in_specs=[pl.BlockSpec((NC, BN, H), lambda n: (0, n, 0))],
        out_specs=pl.BlockSpec((BN, H), lambda n: (n, 0)),
        out_shape=jax.ShapeDtypeStruct((N, H), jnp.float32),
    )(parts)


def kernel(x, edge_index, edge_type, W1, root1, b1, W2, root2, b2, W3, root3, b3):
    src = edge_index[0]
    dst = edge_index[1]
    fidx = edge_type * N + src
    cidx = dst * R + edge_type
    zeros_c = jnp.zeros((CPAD,), jnp.float32)
    zrows = jnp.zeros((NPS, H), jnp.float32)

    inv = _make_cnt_kernel()(cidx.reshape(NS, CCH, EC), zeros_c)
    w = _make_w_kernel()(cidx, inv)

    meta = jnp.stack(
        [fidx.reshape(NW, NCHUNK, EC), dst.reshape(NW, NCHUNK, EC)], axis=2
    )
    w2d = w.reshape(NW, NCHUNK, EC)

    edge_k = _make_edge_kernel()
    h = x
    for (Wr, rootr, br) in ((W1, root1, b1), (W2, root2, b2), (W3, root3, b3)):
        wcat = jnp.concatenate([Wr, rootr[None]], axis=0)
        hcat = _matmul(h, wcat, br.reshape(1, H))
        parts = edge_k(hcat, meta, w2d, zrows)
        h = _combine(parts)
    return h
